# baseline (speedup 1.0000x reference)
import functools

import jax
import jax.numpy as jnp
from jax import lax
from jax.experimental import pallas as pl
from jax.experimental.pallas import tpu as pltpu
from jax.experimental.pallas import tpu_sc as plsc

B = 16384
M = 4096
NW = 32
RPW = B // NW
L = 16
CHROWS = 8
NCH = RPW // CHROWS
NBUF = 3
IN_AHEAD = 1
OUT_WAIT = NBUF - IN_AHEAD


def _sc_body(state_hbm, choice_hbm, out_hbm, choice_v, *rest):
    bufs = list(rest[:NBUF])
    sin = list(rest[NBUF:2 * NBUF])
    sout = list(rest[2 * NBUF:3 * NBUF])
    wid = lax.axis_index("s") * 2 + lax.axis_index("c")
    base = wid * RPW

    lane = lax.iota(jnp.int32, L)
    row_idx = lane & (CHROWS - 1)
    mask = lane < CHROWS
    ones = jnp.full((L,), 1.0, dtype=jnp.float32)

    H = CHROWS // 2

    def start_in(g):
        b = g % NBUF
        r0 = base + g * CHROWS
        h1 = pltpu.async_copy(
            state_hbm.at[pl.ds(r0, H)], bufs[b].at[pl.ds(0, H)], sin[b])
        h2 = pltpu.async_copy(
            state_hbm.at[pl.ds(r0 + H, H)], bufs[b].at[pl.ds(H, H)], sin[b])
        return (h1, h2)

    h_in = {}
    h_out = {}
    for g in range(IN_AHEAD):
        h_in[g] = start_in(g)

    pltpu.sync_copy(choice_hbm.at[pl.ds(base, RPW)], choice_v.at[pl.ds(0, RPW)])
    for g in range(NCH):
        if g >= OUT_WAIT:
            for h in h_out[g - OUT_WAIT]:
                h.wait()
        nxt = g + IN_AHEAD
        if IN_AHEAD <= nxt < NCH:
            h_in[nxt] = start_in(nxt)
        for h in h_in[g]:
            h.wait()
        cvec = choice_v[pl.ds(g * CHROWS, L)] & (M - 1)
        b = g % NBUF
        r0 = base + g * CHROWS
        plsc.addupdate_scatter(bufs[b], [row_idx, cvec], ones, mask=mask)
        ho1 = pltpu.async_copy(
            bufs[b].at[pl.ds(0, H)], out_hbm.at[pl.ds(r0, H)], sout[b])
        ho2 = pltpu.async_copy(
            bufs[b].at[pl.ds(H, H)], out_hbm.at[pl.ds(r0 + H, H)], sout[b])
        h_out[g] = (ho1, ho2)
    for g in range(NCH - OUT_WAIT, NCH):
        for h in h_out[g]:
            h.wait()


_sc_kernel = functools.partial(
    pl.kernel,
    out_type=jax.ShapeDtypeStruct((B, M), jnp.float32),
    mesh=plsc.VectorSubcoreMesh(
        core_axis_name="c", subcore_axis_name="s", num_cores=2, num_subcores=16
    ),
    compiler_params=pltpu.CompilerParams(needs_layout_passes=False),
    scratch_types=(
        [pltpu.VMEM((RPW + L, ), jnp.int32)]
        + [pltpu.VMEM((CHROWS, M), jnp.float32)] * NBUF
        + [pltpu.SemaphoreType.DMA] * (2 * NBUF)
    ),
)(_sc_body)


def kernel(state, choice):
    return _sc_kernel(state, choice.astype(jnp.int32))


# speedup vs baseline: 1.0284x; 1.0284x over previous
import functools

import jax
import jax.numpy as jnp
from jax import lax
from jax.experimental import pallas as pl
from jax.experimental.pallas import tpu as pltpu
from jax.experimental.pallas import tpu_sc as plsc

B = 16384
M = 4096
NW = 32
RPW = B // NW
L = 16
CHROWS = 8
NCH = RPW // CHROWS
NBUF = 3
IN_AHEAD = 1
OUT_WAIT = NBUF - IN_AHEAD


def _sc_body(state_hbm, choice_hbm, out_hbm, choice_v, *rest):
    bufs = list(rest[:NBUF])
    sin = list(rest[NBUF:2 * NBUF])
    sout = list(rest[2 * NBUF:3 * NBUF])
    wid = lax.axis_index("s") * 2 + lax.axis_index("c")
    base = wid * RPW

    lane = lax.iota(jnp.int32, L)
    row_idx = lane & (CHROWS - 1)
    mask = lane < CHROWS
    ones = jnp.full((L,), 1.0, dtype=jnp.float32)

    def start_in(g):
        return pltpu.async_copy(
            state_hbm.at[pl.ds(base + g * CHROWS, CHROWS)],
            bufs[g % NBUF], sin[g % NBUF])

    h_in = {}
    h_out = {}
    for g in range(IN_AHEAD):
        h_in[g] = start_in(g)

    pltpu.sync_copy(choice_hbm.at[pl.ds(base, RPW)], choice_v.at[pl.ds(0, RPW)])
    for g in range(NCH):
        if g >= OUT_WAIT:
            h_out[g - OUT_WAIT].wait()
        nxt = g + IN_AHEAD
        if IN_AHEAD <= nxt < NCH:
            h_in[nxt] = start_in(nxt)
        h_in[g].wait()
        cvec = choice_v[pl.ds(g * CHROWS, L)] & (M - 1)
        b = g % NBUF
        plsc.addupdate_scatter(bufs[b], [row_idx, cvec], ones, mask=mask)
        h_out[g] = pltpu.async_copy(
            bufs[b], out_hbm.at[pl.ds(base + g * CHROWS, CHROWS)], sout[b])
    for g in range(NCH - OUT_WAIT, NCH):
        h_out[g].wait()


_sc_kernel = functools.partial(
    pl.kernel,
    out_type=jax.ShapeDtypeStruct((B, M), jnp.float32),
    mesh=plsc.VectorSubcoreMesh(
        core_axis_name="c", subcore_axis_name="s", num_cores=2, num_subcores=16
    ),
    compiler_params=pltpu.CompilerParams(needs_layout_passes=False),
    scratch_types=(
        [pltpu.VMEM((RPW + L, ), jnp.int32)]
        + [pltpu.VMEM((CHROWS, M), jnp.float32)] * NBUF
        + [pltpu.SemaphoreType.DMA] * (2 * NBUF)
    ),
)(_sc_body)


def kernel(state, choice):
    return _sc_kernel(state, choice.astype(jnp.int32))


# speedup vs baseline: 1.0471x; 1.0182x over previous
import functools

import jax
import jax.numpy as jnp
from jax import lax
from jax.experimental import pallas as pl
from jax.experimental.pallas import tpu as pltpu
from jax.experimental.pallas import tpu_sc as plsc

B = 16384
M = 4096
NW = 32
RPW = B // NW
L = 16
CHROWS = 8
NCH = RPW // CHROWS
NBUF = 3
IN_AHEAD = 1
BUF_WAIT = NBUF - IN_AHEAD
SP_SLABS = 1
SPROWS = CHROWS // 2
SP_MOD = NCH


def _sc_body(state_hbm, choice_hbm, out_hbm, choice_v, *rest):
    bufs = list(rest[:NBUF])
    sin = list(rest[NBUF:2 * NBUF])
    sout = list(rest[2 * NBUF:3 * NBUF])
    s2s = list(rest[3 * NBUF:3 * NBUF + SP_SLABS])
    s2h = list(rest[3 * NBUF + SP_SLABS:3 * NBUF + 2 * SP_SLABS])
    shared = rest[3 * NBUF + 2 * SP_SLABS]
    cid = lax.axis_index("c")
    sid = lax.axis_index("s")
    wid = sid * 2 + cid
    base = wid * RPW

    lane = lax.iota(jnp.int32, L)
    row_idx = lane & (CHROWS - 1)
    mask = lane < CHROWS
    ones = jnp.full((L,), 1.0, dtype=jnp.float32)

    def start_in(g):
        return pltpu.async_copy(
            state_hbm.at[pl.ds(base + g * CHROWS, CHROWS)],
            bufs[g % NBUF], sin[g % NBUF])

    def slab_ref(j):
        row0 = (sid * SP_SLABS + (j % SP_SLABS)) * SPROWS
        return shared.at[pl.ds(row0, SPROWS)]

    h_in = {}
    h_out = {}
    h1 = {}
    h2 = {}
    sp_rows = {}

    for g in range(IN_AHEAD):
        h_in[g] = start_in(g)

    pltpu.sync_copy(choice_hbm.at[pl.ds(base, RPW)], choice_v.at[pl.ds(0, RPW)])

    def flush_pending(jp):
        h1[jp].wait()
        h2[jp] = pltpu.async_copy(
            slab_ref(jp), out_hbm.at[pl.ds(sp_rows[jp], SPROWS)],
            s2h[jp % SP_SLABS])

    k = 0
    pending = ()
    for g in range(NCH):
        for jp in pending:
            flush_pending(jp)
        pending = ()
        gp = g - BUF_WAIT
        if gp >= 0:
            h_out[gp].wait()
        nxt = g + IN_AHEAD
        if IN_AHEAD <= nxt < NCH:
            h_in[nxt] = start_in(nxt)
        h_in[g].wait()
        cvec = choice_v[pl.ds(g * CHROWS, L)] & (M - 1)
        b = g % NBUF
        r0 = base + g * CHROWS
        plsc.addupdate_scatter(bufs[b], [row_idx, cvec], ones, mask=mask)
        if g % SP_MOD != 0:
            if k >= SP_SLABS:
                h2[k - SP_SLABS].wait()
            h1[k] = pltpu.async_copy(
                bufs[b].at[pl.ds(0, SPROWS)], slab_ref(k), s2s[0])
            sp_rows[k] = r0
            pending = (k,)
            k += 1
            h_out[g] = pltpu.async_copy(
                bufs[b].at[pl.ds(SPROWS, SPROWS)],
                out_hbm.at[pl.ds(r0 + SPROWS, SPROWS)], sout[b])
        else:
            h_out[g] = pltpu.async_copy(
                bufs[b], out_hbm.at[pl.ds(r0, CHROWS)], sout[b])
    for jp in pending:
        flush_pending(jp)
    for j in range(max(0, k - SP_SLABS), k):
        h2[j].wait()
    for g in range(max(0, NCH - BUF_WAIT), NCH):
        h_out[g].wait()


_sc_kernel = functools.partial(
    pl.kernel,
    out_type=jax.ShapeDtypeStruct((B, M), jnp.float32),
    mesh=plsc.VectorSubcoreMesh(
        core_axis_name="c", subcore_axis_name="s", num_cores=2, num_subcores=16
    ),
    compiler_params=pltpu.CompilerParams(needs_layout_passes=False),
    scratch_types=(
        [pltpu.VMEM((RPW + L, ), jnp.int32)]
        + [pltpu.VMEM((CHROWS, M), jnp.float32)] * NBUF
        + [pltpu.SemaphoreType.DMA] * (2 * NBUF)
        + [pltpu.SemaphoreType.DMA] * (2 * SP_SLABS)
        + [pltpu.VMEM_SHARED((16 * SP_SLABS * SPROWS, M), jnp.float32)]
    ),
)(_sc_body)


def kernel(state, choice):
    return _sc_kernel(state, choice.astype(jnp.int32))
